# Initial kernel scaffold; baseline (speedup 1.0000x reference)
import jax
import jax.numpy as jnp
from jax.experimental import pallas as pl


def kernel(x_cat, x_cont, tables, W1, b1, g1, be1, W2, b2, g2, be2, Wout, bout, gnum, bnum):
    raise NotImplementedError("write your pallas kernel here")


# speedup vs baseline: 7.0477x; 7.0477x over previous
import functools

import jax
import jax.numpy as jnp
from jax import lax
from jax.experimental import pallas as pl
from jax.experimental.pallas import tpu as pltpu
from jax.experimental.pallas import tpu_sc as plsc

B = 16384
F = 26
V = 100000
D = 32
NNUM = 13
H1 = 512
H2 = 256
ED = F * D
EPS = 1e-5

NC = 2
NS = 16
NW = NC * NS
ROWS = B * F
RPW = ROWS // NW
CHUNK = 1664
NCHUNK = RPW // CHUNK

BM = 1024
NT = B // BM

@functools.lru_cache(maxsize=None)
def _make_sc_gather():
    mesh = plsc.VectorSubcoreMesh(core_axis_name="c", subcore_axis_name="s",
                                  num_cores=NC, num_subcores=NS)

    @functools.partial(
        pl.kernel,
        mesh=mesh,
        out_type=jax.ShapeDtypeStruct((ROWS, D), jnp.float32),
        scratch_types=[
            pltpu.VMEM((RPW,), jnp.int32),
            pltpu.VMEM((CHUNK, D), jnp.float32),
            pltpu.VMEM((CHUNK, D), jnp.float32),
            pltpu.SemaphoreType.DMA,
            pltpu.SemaphoreType.DMA,
        ],
        compiler_params=pltpu.CompilerParams(use_tc_tiling_on_sc=False),
    )
    def _sc_gather(table_hbm, idx_hbm, out_hbm, idx_v, buf0, buf1, sem0, sem1):
        wid = lax.axis_index("s") * NC + lax.axis_index("c")
        base = wid * RPW
        pltpu.sync_copy(idx_hbm.at[pl.ds(base, RPW)], idx_v)
        bufs = (buf0, buf1)
        sems = (sem0, sem1)
        cps = [pltpu.async_copy(table_hbm.at[idx_v.at[pl.ds(0, CHUNK)]],
                                buf0, sem0)]
        for c in range(NCHUNK):
            nxt = c + 1
            if nxt < NCHUNK:
                cps.append(pltpu.async_copy(
                    table_hbm.at[idx_v.at[pl.ds(nxt * CHUNK, CHUNK)]],
                    bufs[nxt % 2], sems[nxt % 2]))
            cps[c].wait()
            pltpu.sync_copy(bufs[c % 2],
                            out_hbm.at[pl.ds(base + c * CHUNK, CHUNK)])

    return _sc_gather


def _gather(table_flat, idx):
    return _make_sc_gather()(table_flat, idx)


def _nt_dot(a, b):
    return lax.dot_general(a, b, (((1,), (1,)), ((), ())),
                           precision=lax.Precision.HIGHEST,
                           preferred_element_type=jnp.float32)


def _k1_body(emb_ref, xc_ref, w1e_ref, w1n_ref, b1_ref, gn_ref, bn_ref,
             h1_ref, st_ref):
    i = pl.program_id(0)
    xc = xc_ref[...]
    mu = jnp.mean(xc, axis=0, keepdims=True)
    ex2 = jnp.mean(xc * xc, axis=0, keepdims=True)
    var = ex2 - mu * mu
    sc = gn_ref[...] * lax.rsqrt(var + EPS)
    sh = bn_ref[...] - mu * sc
    xt = xc_ref[pl.ds(i * BM, BM), :] * sc + sh
    z = _nt_dot(emb_ref[...], w1e_ref[...]) + _nt_dot(xt, w1n_ref[...])
    h = jnp.maximum(z + b1_ref[...], 0.0)
    h1_ref[...] = h

    @pl.when(i == 0)
    def _():
        st_ref[...] = jnp.zeros_like(st_ref)

    st_ref[0:1, :] += jnp.sum(h, axis=0, keepdims=True)
    st_ref[1:2, :] += jnp.sum(h * h, axis=0, keepdims=True)


def _k2_body(h1_ref, st1_ref, w2_ref, b2_ref, g1_ref, be1_ref, h2_ref, st_ref):
    i = pl.program_id(0)
    mu = st1_ref[0:1, :] * (1.0 / B)
    var = st1_ref[1:2, :] * (1.0 / B) - mu * mu
    sc = g1_ref[...] * lax.rsqrt(var + EPS)
    sh = be1_ref[...] - mu * sc
    hn = h1_ref[...] * sc + sh
    h = jnp.maximum(_nt_dot(hn, w2_ref[...]) + b2_ref[...], 0.0)
    h2_ref[...] = h

    @pl.when(i == 0)
    def _():
        st_ref[...] = jnp.zeros_like(st_ref)

    st_ref[0:1, :] += jnp.sum(h, axis=0, keepdims=True)
    st_ref[1:2, :] += jnp.sum(h * h, axis=0, keepdims=True)


def _k3_body(h2_ref, st2_ref, wout_ref, bout_ref, g2_ref, be2_ref, out_ref):
    mu = st2_ref[0:1, :] * (1.0 / B)
    var = st2_ref[1:2, :] * (1.0 / B) - mu * mu
    sc = g2_ref[...] * lax.rsqrt(var + EPS)
    sh = be2_ref[...] - mu * sc
    hn = h2_ref[...] * sc + sh
    z = jnp.sum(hn * wout_ref[...], axis=1, keepdims=True) + bout_ref[...]
    out_ref[...] = 1.0 / (1.0 + jnp.exp(-z))


_k1 = pl.pallas_call(
    _k1_body,
    grid=(NT,),
    in_specs=[
        pl.BlockSpec((BM, ED), lambda i: (i, 0)),
        pl.BlockSpec((B, NNUM), lambda i: (0, 0)),
        pl.BlockSpec((H1, ED), lambda i: (0, 0)),
        pl.BlockSpec((H1, NNUM), lambda i: (0, 0)),
        pl.BlockSpec((1, H1), lambda i: (0, 0)),
        pl.BlockSpec((1, NNUM), lambda i: (0, 0)),
        pl.BlockSpec((1, NNUM), lambda i: (0, 0)),
    ],
    out_specs=[
        pl.BlockSpec((BM, H1), lambda i: (i, 0)),
        pl.BlockSpec((8, H1), lambda i: (0, 0)),
    ],
    out_shape=[
        jax.ShapeDtypeStruct((B, H1), jnp.float32),
        jax.ShapeDtypeStruct((8, H1), jnp.float32),
    ],
)

_k2 = pl.pallas_call(
    _k2_body,
    grid=(NT,),
    in_specs=[
        pl.BlockSpec((BM, H1), lambda i: (i, 0)),
        pl.BlockSpec((8, H1), lambda i: (0, 0)),
        pl.BlockSpec((H2, H1), lambda i: (0, 0)),
        pl.BlockSpec((1, H2), lambda i: (0, 0)),
        pl.BlockSpec((1, H1), lambda i: (0, 0)),
        pl.BlockSpec((1, H1), lambda i: (0, 0)),
    ],
    out_specs=[
        pl.BlockSpec((BM, H2), lambda i: (i, 0)),
        pl.BlockSpec((8, H2), lambda i: (0, 0)),
    ],
    out_shape=[
        jax.ShapeDtypeStruct((B, H2), jnp.float32),
        jax.ShapeDtypeStruct((8, H2), jnp.float32),
    ],
)

_k3 = pl.pallas_call(
    _k3_body,
    grid=(NT,),
    in_specs=[
        pl.BlockSpec((BM, H2), lambda i: (i, 0)),
        pl.BlockSpec((8, H2), lambda i: (0, 0)),
        pl.BlockSpec((1, H2), lambda i: (0, 0)),
        pl.BlockSpec((1, 1), lambda i: (0, 0)),
        pl.BlockSpec((1, H2), lambda i: (0, 0)),
        pl.BlockSpec((1, H2), lambda i: (0, 0)),
    ],
    out_specs=pl.BlockSpec((BM, 1), lambda i: (i, 0)),
    out_shape=jax.ShapeDtypeStruct((B, 1), jnp.float32),
)


def kernel(x_cat, x_cont, tables, W1, b1, g1, be1, W2, b2, g2, be2,
           Wout, bout, gnum, bnum):
    table_flat = tables.reshape(F * V, D)
    idx = (x_cat.astype(jnp.int32)
           + (jnp.arange(F, dtype=jnp.int32) * V)[None, :]).reshape(-1)
    emb = _gather(table_flat, idx).reshape(B, ED)

    w1e = W1[:, :ED]
    w1n = W1[:, ED:]
    h1, st1 = _k1(emb, x_cont, w1e, w1n, b1.reshape(1, H1),
                  gnum.reshape(1, NNUM), bnum.reshape(1, NNUM))
    h2, st2 = _k2(h1, st1, W2, b2.reshape(1, H2),
                  g1.reshape(1, H1), be1.reshape(1, H1))
    out = _k3(h2, st2, Wout.reshape(1, H2), bout.reshape(1, 1),
              g2.reshape(1, H2), be2.reshape(1, H2))
    return out.reshape(B)
